# baseline (speedup 1.0000x reference)
import functools

import jax
import jax.numpy as jnp
from jax.experimental import pallas as pl

_INT_MIN = -2147483648


def _dense_pass_kernel(conf_ref, lab_ref, pred_ref, gt_ref,
                       mining_ref, ce_ref, sl1_ref):
    step = pl.program_id(0)
    x = conf_ref[...]
    tp, c = x.shape
    lab = lab_ref[...]
    m = jnp.max(x, axis=1, keepdims=True)
    s = jnp.sum(jnp.exp(x - m), axis=1, keepdims=True)
    lse = m + jnp.log(s)
    lane = jax.lax.broadcasted_iota(jnp.int32, (tp, c), 1)
    x0 = x[:, 0:1]
    xl = jnp.sum(jnp.where(lane == lab, x, 0.0), axis=1, keepdims=True)
    mining_ref[...] = lse - x0
    ce_ref[...] = lse - xl

    d = pred_ref[...] - gt_ref[...]
    ad = jnp.abs(d)
    sl1 = jnp.where(ad < 1.0, 0.5 * d * d, ad - 0.5)
    row_sl1 = jnp.sum(sl1, axis=1, keepdims=True)
    part = jnp.sum(jnp.where(lab > 0, row_sl1, 0.0))

    @pl.when(step == 0)
    def _():
        sl1_ref[...] = jnp.zeros((1, 1), jnp.float32)

    sl1_ref[...] += part.reshape(1, 1)


def _select_pass_kernel(mining_ref, ce_ref, lab_ref, sl1_ref,
                        reg_ref, cls_ref, tot_ref, *, neg_pos_ratio):
    mining = mining_ref[...]
    lab = lab_ref[...]
    b, p = mining.shape
    pos = lab > 0
    num_pos = jnp.sum(pos.astype(jnp.int32), axis=1, keepdims=True)
    quota = num_pos * neg_pos_ratio
    neg_count = p - num_pos
    take_all = quota >= neg_count

    ib = jax.lax.bitcast_convert_type(mining, jnp.int32)
    key = jnp.where(ib >= 0, ib, ib ^ 0x7FFFFFFF)
    key = jnp.where(pos, jnp.full_like(ib, _INT_MIN), key)

    def vbody(i, t):
        cand = t | jax.lax.shift_left(1, 30 - i)
        cnt = jnp.sum((key >= cand).astype(jnp.int32), axis=1, keepdims=True)
        return jnp.where(cnt >= quota, cand, t)

    t = jax.lax.fori_loop(0, 31, vbody, jnp.zeros((b, 1), jnp.int32))
    count_gt = jnp.sum((key > t).astype(jnp.int32), axis=1, keepdims=True)
    r = quota - count_gt
    tie = key == t
    idx = jax.lax.broadcasted_iota(jnp.int32, (b, p), 1)

    def ibody(i, s):
        cand = s | jax.lax.shift_left(1, 13 - i)
        cnt = jnp.sum((tie & (idx < cand)).astype(jnp.int32),
                      axis=1, keepdims=True)
        return jnp.where(cnt < r, cand, s)

    s = jax.lax.fori_loop(0, 14, ibody, jnp.zeros((b, 1), jnp.int32))

    inc = pos | take_all | (key > t) | (tie & (idx <= s) & (r > 0))
    cls_sum = jnp.sum(jnp.where(inc, ce_ref[...], 0.0))

    np_tot = jnp.sum(num_pos).astype(jnp.float32)
    denom = jnp.maximum(np_tot, 1.0)
    reg = sl1_ref[...] / denom
    cls = (cls_sum / denom).reshape(1, 1)
    reg_ref[...] = reg
    cls_ref[...] = cls
    tot_ref[...] = reg + cls


def kernel(confidence, predicted_locations, gt_locations, labels):
    b, p, c = confidence.shape
    n = b * p
    neg_pos_ratio = 3

    conf2 = confidence.reshape(n, c)
    pred2 = predicted_locations.reshape(n, 4)
    gt2 = gt_locations.reshape(n, 4)
    lab2 = labels.astype(jnp.int32).reshape(n, 1)

    tp = 2048
    grid = (n // tp,)
    mining, ce, sl1 = pl.pallas_call(
        _dense_pass_kernel,
        grid=grid,
        in_specs=[
            pl.BlockSpec((tp, c), lambda i: (i, 0)),
            pl.BlockSpec((tp, 1), lambda i: (i, 0)),
            pl.BlockSpec((tp, 4), lambda i: (i, 0)),
            pl.BlockSpec((tp, 4), lambda i: (i, 0)),
        ],
        out_specs=[
            pl.BlockSpec((tp, 1), lambda i: (i, 0)),
            pl.BlockSpec((tp, 1), lambda i: (i, 0)),
            pl.BlockSpec((1, 1), lambda i: (0, 0)),
        ],
        out_shape=[
            jax.ShapeDtypeStruct((n, 1), jnp.float32),
            jax.ShapeDtypeStruct((n, 1), jnp.float32),
            jax.ShapeDtypeStruct((1, 1), jnp.float32),
        ],
    )(conf2, lab2, pred2, gt2)

    reg, cls, tot = pl.pallas_call(
        functools.partial(_select_pass_kernel, neg_pos_ratio=neg_pos_ratio),
        in_specs=[
            pl.BlockSpec((b, p), lambda: (0, 0)),
            pl.BlockSpec((b, p), lambda: (0, 0)),
            pl.BlockSpec((b, p), lambda: (0, 0)),
            pl.BlockSpec((1, 1), lambda: (0, 0)),
        ],
        out_specs=[
            pl.BlockSpec((1, 1), lambda: (0, 0)),
            pl.BlockSpec((1, 1), lambda: (0, 0)),
            pl.BlockSpec((1, 1), lambda: (0, 0)),
        ],
        out_shape=[
            jax.ShapeDtypeStruct((1, 1), jnp.float32),
            jax.ShapeDtypeStruct((1, 1), jnp.float32),
            jax.ShapeDtypeStruct((1, 1), jnp.float32),
        ],
    )(mining.reshape(b, p), ce.reshape(b, p),
      labels.astype(jnp.int32), sl1)

    return (reg[0, 0], cls[0, 0], tot[0, 0])
